# baseline (speedup 1.0000x reference)
import functools

import jax
import jax.numpy as jnp
from jax import lax
from jax.experimental import pallas as pl
from jax.experimental.pallas import tpu as pltpu
from jax.experimental.pallas import tpu_sc as plsc

_NC = 2
_NS = 16
_LANES = 16


@functools.lru_cache(maxsize=None)
def _make_scatter(n, d, e_pad):
    nw = _NC * _NS
    chunk = 128
    assert e_pad % (nw * chunk) == 0
    cpw = e_pad // (nw * chunk)
    epw = cpw * chunk
    assert cpw % 16 == 0 and epw % 8 == 0
    hcp = cpw // 2
    rpt = (n // _NS) // 8 * 8
    rextra = n - _NS * rpt
    assert rextra % 8 == 0
    zsrc = min(chunk, rpt)
    nacc = n + 128
    mesh = plsc.VectorSubcoreMesh(core_axis_name="c", subcore_axis_name="s")

    @functools.partial(
        pl.kernel,
        mesh=mesh,
        out_type=jax.ShapeDtypeStruct((_NC, n, d), jnp.float32),
        scratch_types=[
            pltpu.VMEM((chunk,), jnp.int32),
            pltpu.VMEM((chunk,), jnp.int32),
            pltpu.VMEM((chunk,), jnp.int32),
            pltpu.VMEM((chunk,), jnp.int32),
            pltpu.VMEM((chunk, d), jnp.float32),
            pltpu.VMEM((chunk, d), jnp.float32),
            pltpu.SemaphoreType.DMA,
            pltpu.SemaphoreType.DMA,
            pltpu.SemaphoreType.DMA,
            pltpu.SemaphoreType.DMA,
            pltpu.SemaphoreType.DMA,
            pltpu.SemaphoreType.DMA,
            pltpu.VMEM_SHARED((nacc, d), jnp.float32),
        ],
    )
    def scatter_kernel(h_hbm, src_hbm, dst_hbm, out_hbm,
                       sA, sB, dA, dB, r0, r1,
                       isemA, isemB, gsemA, gsemB, ssemA, ssemB, acc_sh):
        srcv = (sA, sB)
        dstv = (dA, dB)
        rows = (r0, r1)
        isem = (isemA, isemB)
        gsem = (gsemA, gsemB)
        ssem = (ssemA, ssemB)
        cid = lax.axis_index("c")
        sid = lax.axis_index("s")
        wid = sid * _NC + cid

        def zstore(t, carry):
            r = t // (d // _LANES)
            c16 = (t % (d // _LANES)) * _LANES
            r0[r, pl.ds(c16, _LANES)] = jnp.zeros((_LANES,), jnp.float32)
            return carry
        lax.fori_loop(0, zsrc * (d // _LANES), zstore, 0)
        row0 = sid * rpt
        nz_full = rpt // zsrc
        def zcopy(j, carry):
            pltpu.sync_copy(r0.at[pl.ds(0, zsrc)],
                            acc_sh.at[pl.ds(row0 + j * zsrc, zsrc)])
            return carry
        lax.fori_loop(0, nz_full, zcopy, 0)
        zrem = rpt - nz_full * zsrc
        if zrem:
            pltpu.sync_copy(r0.at[pl.ds(0, zrem)],
                            acc_sh.at[pl.ds(row0 + nz_full * zsrc, zrem)])
        @pl.when(sid == _NS - 1)
        def _ztail():
            pltpu.sync_copy(r0.at[pl.ds(0, rextra)],
                            acc_sh.at[pl.ds(_NS * rpt, rextra)])
        plsc.subcore_barrier()

        e0 = wid * cpw * chunk
        def step(i, carry):
            hidx = []
            for b in range(2):
                off = e0 + (2 * i + b) * chunk
                hidx.append(pltpu.async_copy(
                    src_hbm.at[pl.ds(off, chunk)], srcv[b], isem[b]))
                hidx.append(pltpu.async_copy(
                    dst_hbm.at[pl.ds(off, chunk)], dstv[b], isem[b]))
            hg = []
            for b in range(2):
                hidx[2 * b].wait()
                hidx[2 * b + 1].wait()
                hg.append(pltpu.async_copy(h_hbm.at[srcv[b]], rows[b], gsem[b]))
            hs = []
            for b in range(2):
                hg[b].wait()
                hs.append(pltpu.async_copy(rows[b], acc_sh.at[dstv[b]],
                                           ssem[b], add=True))
            for b in range(2):
                hs[b].wait()
            return carry
        lax.fori_loop(0, cpw // 2, step, 0)
        plsc.subcore_barrier()

        pltpu.sync_copy(acc_sh.at[pl.ds(row0, rpt)], out_hbm.at[cid].at[pl.ds(row0, rpt)])
        @pl.when(sid == _NS - 1)
        def _():
            pltpu.sync_copy(acc_sh.at[pl.ds(_NS * rpt, rextra)],
                            out_hbm.at[cid].at[pl.ds(_NS * rpt, rextra)])

    return scatter_kernel


@functools.lru_cache(maxsize=None)
def _make_dense(n, d_in, d, relu_out):
    def body(h_ref, a0_ref, a1_ref, w1_ref, b1_ref, w2_ref, b2_ref,
             g_ref, bt_ref, o_ref):
        z = h_ref[...] + a0_ref[...] + a1_ref[...]
        z = jnp.dot(z, w1_ref[...], preferred_element_type=jnp.float32) + b1_ref[...]
        z = jnp.maximum(z, 0.0)
        z = jnp.dot(z, w2_ref[...], preferred_element_type=jnp.float32) + b2_ref[...]
        mu = jnp.mean(z, axis=0, keepdims=True)
        var = jnp.mean((z - mu) * (z - mu), axis=0, keepdims=True)
        z = g_ref[...] * (z - mu) * lax.rsqrt(var + 1e-5) + bt_ref[...]
        if relu_out:
            z = jnp.maximum(z, 0.0)
        o_ref[...] = z

    return pl.pallas_call(
        body,
        out_shape=jax.ShapeDtypeStruct((n, d), jnp.float32),
    )


def kernel(nodes, edge_indexs, graph_indicators,
           W1_0, b1_0, W2_0, b2_0, gamma_0, beta_0,
           W1_1, b1_1, W2_1, b2_1, gamma_1, beta_1,
           W1_2, b1_2, W2_2, b2_2, gamma_2, beta_2):
    del graph_indicators
    n, d = nodes.shape
    e = edge_indexs.shape[1]
    chunk, nw = 128, _NC * _NS
    q = -(-e // (nw * chunk))
    cpw = -(-q // 16) * 16
    e_pad = nw * cpw * chunk
    npad = e_pad - e
    src = jnp.concatenate([edge_indexs[0], jnp.zeros((npad,), jnp.int32)])
    dst = jnp.concatenate(
        [edge_indexs[1], n + jnp.arange(npad, dtype=jnp.int32) % 128])
    params = [
        (W1_0, b1_0, W2_0, b2_0, gamma_0, beta_0),
        (W1_1, b1_1, W2_1, b2_1, gamma_1, beta_1),
        (W1_2, b1_2, W2_2, b2_2, gamma_2, beta_2),
    ]
    scatter = _make_scatter(n, d, e_pad)
    h = nodes
    for layer, (w1, b1, w2, b2, g, bt) in enumerate(params):
        agg = scatter(h, src, dst)
        dense = _make_dense(n, w1.shape[0], d, layer < len(params) - 1)
        h = dense(h, agg[0], agg[1], w1, b1.reshape(1, d), w2, b2.reshape(1, d),
                  g.reshape(1, d), bt.reshape(1, d))
    return h


# speedup vs baseline: 2.0645x; 2.0645x over previous
import functools

import jax
import jax.numpy as jnp
from jax import lax
from jax.experimental import pallas as pl
from jax.experimental.pallas import tpu as pltpu
from jax.experimental.pallas import tpu_sc as plsc

_NC = 2
_NS = 16
_LANES = 16


@functools.lru_cache(maxsize=None)
def _make_scatter(n, d, e):
    nw = _NC * _NS
    assert e % nw == 0
    epw = e // nw
    chunk = 128
    full = epw // chunk
    tail = epw % chunk
    assert tail % 8 == 0
    rpt = (n // _NS) // 8 * 8
    rextra = n - _NS * rpt
    assert rextra % 8 == 0
    zrows = 208
    assert rpt % zrows == 0 and rextra <= zrows
    mesh = plsc.VectorSubcoreMesh(core_axis_name="c", subcore_axis_name="s")

    @functools.partial(
        pl.kernel,
        mesh=mesh,
        out_type=jax.ShapeDtypeStruct((_NC, n, d), jnp.float32),
        scratch_types=[
            pltpu.VMEM((zrows, d), jnp.float32),
            pltpu.VMEM((chunk,), jnp.int32),
            pltpu.VMEM((chunk,), jnp.int32),
            pltpu.VMEM((chunk, d), jnp.float32),
            pltpu.VMEM_SHARED((n, d), jnp.float32),
            pltpu.SemaphoreType.DMA,
        ],
    )
    def scatter_kernel(h_hbm, src_hbm, dst_hbm, out_hbm,
                       zbuf, src_v, dst_v, rows_v, acc_sh, sem):
        cid = lax.axis_index("c")
        sid = lax.axis_index("s")
        wid = sid * _NC + cid

        def zstore(t, carry):
            r = t // (d // _LANES)
            c16 = (t % (d // _LANES)) * _LANES
            zbuf[r, pl.ds(c16, _LANES)] = jnp.zeros((_LANES,), jnp.float32)
            return carry
        lax.fori_loop(0, zrows * (d // _LANES), zstore, 0)
        row0 = sid * rpt
        def zcopy(j, carry):
            pltpu.sync_copy(zbuf, acc_sh.at[pl.ds(row0 + j * zrows, zrows)])
            return carry
        lax.fori_loop(0, rpt // zrows, zcopy, 0)
        @pl.when(sid == _NS - 1)
        def _ztail():
            pltpu.sync_copy(zbuf.at[pl.ds(0, rextra)],
                            acc_sh.at[pl.ds(_NS * rpt, rextra)])
        plsc.subcore_barrier()

        e0 = wid * epw
        def body(t, carry):
            base = e0 + t * chunk
            pltpu.sync_copy(src_hbm.at[pl.ds(base, chunk)], src_v)
            pltpu.sync_copy(dst_hbm.at[pl.ds(base, chunk)], dst_v)
            pltpu.async_copy(h_hbm.at[src_v], rows_v, sem).wait()
            pltpu.sync_copy(rows_v, acc_sh.at[dst_v], add=True)
            return carry
        lax.fori_loop(0, full, body, 0)
        if tail:
            base = e0 + full * chunk
            pltpu.sync_copy(src_hbm.at[pl.ds(base, tail)], src_v.at[pl.ds(0, tail)])
            pltpu.sync_copy(dst_hbm.at[pl.ds(base, tail)], dst_v.at[pl.ds(0, tail)])
            pltpu.async_copy(h_hbm.at[src_v.at[pl.ds(0, tail)]],
                             rows_v.at[pl.ds(0, tail)], sem).wait()
            pltpu.sync_copy(rows_v.at[pl.ds(0, tail)],
                            acc_sh.at[dst_v.at[pl.ds(0, tail)]], add=True)
        plsc.subcore_barrier()

        pltpu.sync_copy(acc_sh.at[pl.ds(row0, rpt)], out_hbm.at[cid].at[pl.ds(row0, rpt)])
        @pl.when(sid == _NS - 1)
        def _otail():
            pltpu.sync_copy(acc_sh.at[pl.ds(_NS * rpt, rextra)],
                            out_hbm.at[cid].at[pl.ds(_NS * rpt, rextra)])

    return scatter_kernel


@functools.lru_cache(maxsize=None)
def _make_dense(n, d_in, d, relu_out):
    def body(h_ref, a0_ref, a1_ref, w1_ref, b1_ref, w2_ref, b2_ref,
             g_ref, bt_ref, o_ref):
        z = h_ref[...] + a0_ref[...] + a1_ref[...]
        z = jnp.dot(z, w1_ref[...], preferred_element_type=jnp.float32) + b1_ref[...]
        z = jnp.maximum(z, 0.0)
        z = jnp.dot(z, w2_ref[...], preferred_element_type=jnp.float32) + b2_ref[...]
        mu = jnp.mean(z, axis=0, keepdims=True)
        var = jnp.mean((z - mu) * (z - mu), axis=0, keepdims=True)
        z = g_ref[...] * (z - mu) * lax.rsqrt(var + 1e-5) + bt_ref[...]
        if relu_out:
            z = jnp.maximum(z, 0.0)
        o_ref[...] = z

    return pl.pallas_call(
        body,
        out_shape=jax.ShapeDtypeStruct((n, d), jnp.float32),
    )


def kernel(nodes, edge_indexs, graph_indicators,
           W1_0, b1_0, W2_0, b2_0, gamma_0, beta_0,
           W1_1, b1_1, W2_1, b2_1, gamma_1, beta_1,
           W1_2, b1_2, W2_2, b2_2, gamma_2, beta_2):
    del graph_indicators
    n, d = nodes.shape
    e = edge_indexs.shape[1]
    src = edge_indexs[0]
    dst = edge_indexs[1]
    params = [
        (W1_0, b1_0, W2_0, b2_0, gamma_0, beta_0),
        (W1_1, b1_1, W2_1, b2_1, gamma_1, beta_1),
        (W1_2, b1_2, W2_2, b2_2, gamma_2, beta_2),
    ]
    scatter = _make_scatter(n, d, e)
    h = nodes
    for layer, (w1, b1, w2, b2, g, bt) in enumerate(params):
        agg = scatter(h, src, dst)
        dense = _make_dense(n, w1.shape[0], d, layer < len(params) - 1)
        h = dense(h, agg[0], agg[1], w1, b1.reshape(1, d), w2, b2.reshape(1, d),
                  g.reshape(1, d), bt.reshape(1, d))
    return h


# speedup vs baseline: 2.3481x; 1.1374x over previous
import functools

import jax
import jax.numpy as jnp
from jax import lax
from jax.experimental import pallas as pl
from jax.experimental.pallas import tpu as pltpu
from jax.experimental.pallas import tpu_sc as plsc

_NC = 2
_NS = 16
_LANES = 16


@functools.lru_cache(maxsize=None)
def _make_scatter(n, d, e):
    nw = _NC * _NS
    assert e % nw == 0
    epw = e // nw
    chunk = 128
    full = epw // chunk
    tail = epw % chunk
    assert tail % 8 == 0
    rpt = (n // _NS) // 8 * 8
    rextra = n - _NS * rpt
    assert rextra % 8 == 0
    zrows = 208
    assert rpt % zrows == 0 and rextra <= zrows
    mesh = plsc.VectorSubcoreMesh(core_axis_name="c", subcore_axis_name="s")

    @functools.partial(
        pl.kernel,
        mesh=mesh,
        out_type=jax.ShapeDtypeStruct((_NC, n, d), jnp.float32),
        scratch_types=[
            pltpu.VMEM((zrows, d), jnp.float32),
            pltpu.VMEM((chunk,), jnp.int32),
            pltpu.VMEM((chunk,), jnp.int32),
            pltpu.VMEM((chunk, d), jnp.float32),
            pltpu.VMEM_SHARED((n, d), jnp.float32),
            pltpu.SemaphoreType.DMA,
        ],
    )
    def scatter_kernel(h_hbm, src_hbm, dst_hbm, out_hbm,
                       zbuf, src_v, dst_v, rows_v, acc_sh, sem):
        cid = lax.axis_index("c")
        sid = lax.axis_index("s")
        wid = sid * _NC + cid

        def zstore(t, carry):
            r = t // (d // _LANES)
            c16 = (t % (d // _LANES)) * _LANES
            zbuf[r, pl.ds(c16, _LANES)] = jnp.zeros((_LANES,), jnp.float32)
            return carry
        lax.fori_loop(0, zrows * (d // _LANES), zstore, 0)
        row0 = sid * rpt
        def zcopy(j, carry):
            pltpu.sync_copy(zbuf, acc_sh.at[pl.ds(row0 + j * zrows, zrows)])
            return carry
        lax.fori_loop(0, rpt // zrows, zcopy, 0)
        @pl.when(sid == _NS - 1)
        def _ztail():
            pltpu.sync_copy(zbuf.at[pl.ds(0, rextra)],
                            acc_sh.at[pl.ds(_NS * rpt, rextra)])
        plsc.subcore_barrier()

        e0 = wid * epw
        def body(t, carry):
            base = e0 + t * chunk
            h1 = pltpu.async_copy(src_hbm.at[pl.ds(base, chunk)], src_v, sem)
            h2 = pltpu.async_copy(dst_hbm.at[pl.ds(base, chunk)], dst_v, sem)
            h1.wait()
            h2.wait()
            pltpu.async_copy(h_hbm.at[src_v], rows_v, sem).wait()
            pltpu.sync_copy(rows_v, acc_sh.at[dst_v], add=True)
            return carry
        lax.fori_loop(0, full, body, 0)
        if tail:
            base = e0 + full * chunk
            pltpu.sync_copy(src_hbm.at[pl.ds(base, tail)], src_v.at[pl.ds(0, tail)])
            pltpu.sync_copy(dst_hbm.at[pl.ds(base, tail)], dst_v.at[pl.ds(0, tail)])
            pltpu.async_copy(h_hbm.at[src_v.at[pl.ds(0, tail)]],
                             rows_v.at[pl.ds(0, tail)], sem).wait()
            pltpu.sync_copy(rows_v.at[pl.ds(0, tail)],
                            acc_sh.at[dst_v.at[pl.ds(0, tail)]], add=True)
        plsc.subcore_barrier()

        pltpu.sync_copy(acc_sh.at[pl.ds(row0, rpt)], out_hbm.at[cid].at[pl.ds(row0, rpt)])
        @pl.when(sid == _NS - 1)
        def _otail():
            pltpu.sync_copy(acc_sh.at[pl.ds(_NS * rpt, rextra)],
                            out_hbm.at[cid].at[pl.ds(_NS * rpt, rextra)])

    return scatter_kernel


@functools.lru_cache(maxsize=None)
def _make_dense(n, d_in, d, relu_out):
    def body(h_ref, a0_ref, a1_ref, w1_ref, b1_ref, w2_ref, b2_ref,
             g_ref, bt_ref, o_ref):
        z = h_ref[...] + a0_ref[...] + a1_ref[...]
        z = jnp.dot(z, w1_ref[...], preferred_element_type=jnp.float32) + b1_ref[...]
        z = jnp.maximum(z, 0.0)
        z = jnp.dot(z, w2_ref[...], preferred_element_type=jnp.float32) + b2_ref[...]
        mu = jnp.mean(z, axis=0, keepdims=True)
        var = jnp.mean((z - mu) * (z - mu), axis=0, keepdims=True)
        z = g_ref[...] * (z - mu) * lax.rsqrt(var + 1e-5) + bt_ref[...]
        if relu_out:
            z = jnp.maximum(z, 0.0)
        o_ref[...] = z

    return pl.pallas_call(
        body,
        out_shape=jax.ShapeDtypeStruct((n, d), jnp.float32),
    )


def kernel(nodes, edge_indexs, graph_indicators,
           W1_0, b1_0, W2_0, b2_0, gamma_0, beta_0,
           W1_1, b1_1, W2_1, b2_1, gamma_1, beta_1,
           W1_2, b1_2, W2_2, b2_2, gamma_2, beta_2):
    del graph_indicators
    n, d = nodes.shape
    e = edge_indexs.shape[1]
    src = edge_indexs[0]
    dst = edge_indexs[1]
    params = [
        (W1_0, b1_0, W2_0, b2_0, gamma_0, beta_0),
        (W1_1, b1_1, W2_1, b2_1, gamma_1, beta_1),
        (W1_2, b1_2, W2_2, b2_2, gamma_2, beta_2),
    ]
    scatter = _make_scatter(n, d, e)
    h = nodes
    for layer, (w1, b1, w2, b2, g, bt) in enumerate(params):
        agg = scatter(h, src, dst)
        dense = _make_dense(n, w1.shape[0], d, layer < len(params) - 1)
        h = dense(h, agg[0], agg[1], w1, b1.reshape(1, d), w2, b2.reshape(1, d),
                  g.reshape(1, d), bt.reshape(1, d))
    return h


# speedup vs baseline: 2.9070x; 1.2380x over previous
import functools

import jax
import jax.numpy as jnp
from jax import lax
from jax.experimental import pallas as pl
from jax.experimental.pallas import tpu as pltpu
from jax.experimental.pallas import tpu_sc as plsc

_NC = 2
_NS = 16
_LANES = 16


@functools.lru_cache(maxsize=None)
def _make_scatter(n, d, e):
    nw = _NC * _NS
    assert e % nw == 0
    epw = e // nw
    chunk = 128
    full = epw // chunk
    tail = epw % chunk
    assert tail % 8 == 0
    rpt = (n // _NS) // 8 * 8
    rextra = n - _NS * rpt
    assert rextra % 8 == 0
    zrows = 48
    assert rpt % zrows == 0 and rextra <= zrows
    assert full % 2 == 0
    mesh = plsc.VectorSubcoreMesh(core_axis_name="c", subcore_axis_name="s")

    @functools.partial(
        pl.kernel,
        mesh=mesh,
        out_type=jax.ShapeDtypeStruct((_NC, n, d), jnp.float32),
        scratch_types=[
            pltpu.VMEM((zrows, d), jnp.float32),
            pltpu.VMEM((chunk,), jnp.int32),
            pltpu.VMEM((chunk,), jnp.int32),
            pltpu.VMEM((chunk,), jnp.int32),
            pltpu.VMEM((chunk,), jnp.int32),
            pltpu.VMEM((chunk, d), jnp.float32),
            pltpu.VMEM((chunk, d), jnp.float32),
            pltpu.VMEM_SHARED((n, d), jnp.float32),
            pltpu.SemaphoreType.DMA,
            pltpu.SemaphoreType.DMA,
            pltpu.SemaphoreType.DMA,
            pltpu.SemaphoreType.DMA,
        ],
    )
    def scatter_kernel(h_hbm, src_hbm, dst_hbm, out_hbm,
                       zbuf, sv0, sv1, dv0, dv1, rv0, rv1, acc_sh,
                       isem0, isem1, gsem0, gsem1):
        src_v, dst_v, rows_v, sem = sv0, dv0, rv0, gsem0
        srcv = (sv0, sv1)
        dstv = (dv0, dv1)
        rowsv = (rv0, rv1)
        isem = (isem0, isem1)
        gsem = (gsem0, gsem1)
        cid = lax.axis_index("c")
        sid = lax.axis_index("s")
        wid = sid * _NC + cid

        def zstore(t, carry):
            r = t // (d // _LANES)
            c16 = (t % (d // _LANES)) * _LANES
            zbuf[r, pl.ds(c16, _LANES)] = jnp.zeros((_LANES,), jnp.float32)
            return carry
        lax.fori_loop(0, zrows * (d // _LANES), zstore, 0)
        row0 = sid * rpt
        def zcopy(j, carry):
            pltpu.sync_copy(zbuf, acc_sh.at[pl.ds(row0 + j * zrows, zrows)])
            return carry
        lax.fori_loop(0, rpt // zrows, zcopy, 0)
        @pl.when(sid == _NS - 1)
        def _ztail():
            pltpu.sync_copy(zbuf.at[pl.ds(0, rextra)],
                            acc_sh.at[pl.ds(_NS * rpt, rextra)])
        plsc.subcore_barrier()

        e0 = wid * epw
        def body(i, carry):
            hi = []
            for b in range(2):
                base = e0 + (2 * i + b) * chunk
                hi.append(pltpu.async_copy(
                    src_hbm.at[pl.ds(base, chunk)], srcv[b], isem[b]))
                hi.append(pltpu.async_copy(
                    dst_hbm.at[pl.ds(base, chunk)], dstv[b], isem[b]))
            hg = []
            for b in range(2):
                hi[2 * b].wait()
                hi[2 * b + 1].wait()
                hg.append(pltpu.async_copy(h_hbm.at[srcv[b]], rowsv[b], gsem[b]))
            for b in range(2):
                hg[b].wait()
                pltpu.sync_copy(rowsv[b], acc_sh.at[dstv[b]], add=True)
            return carry
        lax.fori_loop(0, full // 2, body, 0)
        if tail:
            base = e0 + full * chunk
            pltpu.sync_copy(src_hbm.at[pl.ds(base, tail)], src_v.at[pl.ds(0, tail)])
            pltpu.sync_copy(dst_hbm.at[pl.ds(base, tail)], dst_v.at[pl.ds(0, tail)])
            pltpu.async_copy(h_hbm.at[src_v.at[pl.ds(0, tail)]],
                             rows_v.at[pl.ds(0, tail)], sem).wait()
            pltpu.sync_copy(rows_v.at[pl.ds(0, tail)],
                            acc_sh.at[dst_v.at[pl.ds(0, tail)]], add=True)
        plsc.subcore_barrier()

        pltpu.sync_copy(acc_sh.at[pl.ds(row0, rpt)], out_hbm.at[cid].at[pl.ds(row0, rpt)])
        @pl.when(sid == _NS - 1)
        def _otail():
            pltpu.sync_copy(acc_sh.at[pl.ds(_NS * rpt, rextra)],
                            out_hbm.at[cid].at[pl.ds(_NS * rpt, rextra)])

    return scatter_kernel


@functools.lru_cache(maxsize=None)
def _make_dense(n, d_in, d, relu_out):
    def body(h_ref, a0_ref, a1_ref, w1_ref, b1_ref, w2_ref, b2_ref,
             g_ref, bt_ref, o_ref):
        z = h_ref[...] + a0_ref[...] + a1_ref[...]
        z = jnp.dot(z, w1_ref[...], preferred_element_type=jnp.float32) + b1_ref[...]
        z = jnp.maximum(z, 0.0)
        z = jnp.dot(z, w2_ref[...], preferred_element_type=jnp.float32) + b2_ref[...]
        mu = jnp.mean(z, axis=0, keepdims=True)
        var = jnp.mean((z - mu) * (z - mu), axis=0, keepdims=True)
        z = g_ref[...] * (z - mu) * lax.rsqrt(var + 1e-5) + bt_ref[...]
        if relu_out:
            z = jnp.maximum(z, 0.0)
        o_ref[...] = z

    return pl.pallas_call(
        body,
        out_shape=jax.ShapeDtypeStruct((n, d), jnp.float32),
    )


def kernel(nodes, edge_indexs, graph_indicators,
           W1_0, b1_0, W2_0, b2_0, gamma_0, beta_0,
           W1_1, b1_1, W2_1, b2_1, gamma_1, beta_1,
           W1_2, b1_2, W2_2, b2_2, gamma_2, beta_2):
    del graph_indicators
    n, d = nodes.shape
    e = edge_indexs.shape[1]
    src = edge_indexs[0]
    dst = edge_indexs[1]
    params = [
        (W1_0, b1_0, W2_0, b2_0, gamma_0, beta_0),
        (W1_1, b1_1, W2_1, b2_1, gamma_1, beta_1),
        (W1_2, b1_2, W2_2, b2_2, gamma_2, beta_2),
    ]
    scatter = _make_scatter(n, d, e)
    h = nodes
    for layer, (w1, b1, w2, b2, g, bt) in enumerate(params):
        agg = scatter(h, src, dst)
        dense = _make_dense(n, w1.shape[0], d, layer < len(params) - 1)
        h = dense(h, agg[0], agg[1], w1, b1.reshape(1, d), w2, b2.reshape(1, d),
                  g.reshape(1, d), bt.reshape(1, d))
    return h


# speedup vs baseline: 2.9350x; 1.0097x over previous
import functools

import jax
import jax.numpy as jnp
from jax import lax
from jax.experimental import pallas as pl
from jax.experimental.pallas import tpu as pltpu
from jax.experimental.pallas import tpu_sc as plsc

_NC = 2
_NS = 16
_LANES = 16


@functools.lru_cache(maxsize=None)
def _make_scatter(n, d, e):
    nw = _NC * _NS
    assert e % nw == 0
    epw = e // nw
    chunk = 128
    full = epw // chunk
    tail = epw % chunk
    assert tail % 8 == 0
    rpt = (n // _NS) // 8 * 8
    rextra = n - _NS * rpt
    assert rextra % 8 == 0
    zrows = 48
    assert rpt % zrows == 0 and rextra <= zrows
    assert full % 2 == 0
    mesh = plsc.VectorSubcoreMesh(core_axis_name="c", subcore_axis_name="s")

    @functools.partial(
        pl.kernel,
        mesh=mesh,
        out_type=jax.ShapeDtypeStruct((_NC, n, d), jnp.float32),
        scratch_types=[
            pltpu.VMEM((zrows, d), jnp.float32),
            pltpu.VMEM((chunk,), jnp.int32),
            pltpu.VMEM((chunk,), jnp.int32),
            pltpu.VMEM((chunk,), jnp.int32),
            pltpu.VMEM((chunk,), jnp.int32),
            pltpu.VMEM((chunk, d), jnp.float32),
            pltpu.VMEM((chunk, d), jnp.float32),
            pltpu.VMEM_SHARED((n, d), jnp.float32),
            pltpu.SemaphoreType.DMA,
            pltpu.SemaphoreType.DMA,
            pltpu.SemaphoreType.DMA,
            pltpu.SemaphoreType.DMA,
            pltpu.SemaphoreType.DMA,
            pltpu.SemaphoreType.DMA,
        ],
    )
    def scatter_kernel(h_hbm, src_hbm, dst_hbm, out_hbm,
                       zbuf, sv0, sv1, dv0, dv1, rv0, rv1, acc_sh,
                       isem0, isem1, gsem0, gsem1, ssem0, ssem1):
        src_v, dst_v, rows_v, sem = sv0, dv0, rv0, gsem0
        srcv = (sv0, sv1)
        dstv = (dv0, dv1)
        rowsv = (rv0, rv1)
        isem = (isem0, isem1)
        gsem = (gsem0, gsem1)
        ssem = (ssem0, ssem1)
        cid = lax.axis_index("c")
        sid = lax.axis_index("s")
        wid = sid * _NC + cid

        def zstore(t, carry):
            r = t // (d // _LANES)
            c16 = (t % (d // _LANES)) * _LANES
            zbuf[r, pl.ds(c16, _LANES)] = jnp.zeros((_LANES,), jnp.float32)
            return carry
        lax.fori_loop(0, zrows * (d // _LANES), zstore, 0)
        row0 = sid * rpt
        def zcopy(j, carry):
            pltpu.sync_copy(zbuf, acc_sh.at[pl.ds(row0 + j * zrows, zrows)])
            return carry
        lax.fori_loop(0, rpt // zrows, zcopy, 0)
        @pl.when(sid == _NS - 1)
        def _ztail():
            pltpu.sync_copy(zbuf.at[pl.ds(0, rextra)],
                            acc_sh.at[pl.ds(_NS * rpt, rextra)])
        plsc.subcore_barrier()

        e0 = wid * epw
        def body(i, carry):
            hi = []
            for b in range(2):
                base = e0 + (2 * i + b) * chunk
                hi.append(pltpu.async_copy(
                    src_hbm.at[pl.ds(base, chunk)], srcv[b], isem[b]))
                hi.append(pltpu.async_copy(
                    dst_hbm.at[pl.ds(base, chunk)], dstv[b], isem[b]))
            hg = []
            for b in range(2):
                hi[2 * b].wait()
                hi[2 * b + 1].wait()
                hg.append(pltpu.async_copy(h_hbm.at[srcv[b]], rowsv[b], gsem[b]))
            hs = []
            for b in range(2):
                hg[b].wait()
                hs.append(pltpu.async_copy(rowsv[b], acc_sh.at[dstv[b]],
                                           ssem[b], add=True))
            for b in range(2):
                hs[b].wait()
            return carry
        lax.fori_loop(0, full // 2, body, 0)
        if tail:
            base = e0 + full * chunk
            pltpu.sync_copy(src_hbm.at[pl.ds(base, tail)], src_v.at[pl.ds(0, tail)])
            pltpu.sync_copy(dst_hbm.at[pl.ds(base, tail)], dst_v.at[pl.ds(0, tail)])
            pltpu.async_copy(h_hbm.at[src_v.at[pl.ds(0, tail)]],
                             rows_v.at[pl.ds(0, tail)], sem).wait()
            pltpu.sync_copy(rows_v.at[pl.ds(0, tail)],
                            acc_sh.at[dst_v.at[pl.ds(0, tail)]], add=True)
        plsc.subcore_barrier()

        pltpu.sync_copy(acc_sh.at[pl.ds(row0, rpt)], out_hbm.at[cid].at[pl.ds(row0, rpt)])
        @pl.when(sid == _NS - 1)
        def _otail():
            pltpu.sync_copy(acc_sh.at[pl.ds(_NS * rpt, rextra)],
                            out_hbm.at[cid].at[pl.ds(_NS * rpt, rextra)])

    return scatter_kernel


@functools.lru_cache(maxsize=None)
def _make_dense(n, d_in, d, relu_out):
    def body(h_ref, a0_ref, a1_ref, w1_ref, b1_ref, w2_ref, b2_ref,
             g_ref, bt_ref, o_ref):
        z = h_ref[...] + a0_ref[...] + a1_ref[...]
        z = jnp.dot(z, w1_ref[...], preferred_element_type=jnp.float32) + b1_ref[...]
        z = jnp.maximum(z, 0.0)
        z = jnp.dot(z, w2_ref[...], preferred_element_type=jnp.float32) + b2_ref[...]
        mu = jnp.mean(z, axis=0, keepdims=True)
        var = jnp.mean((z - mu) * (z - mu), axis=0, keepdims=True)
        z = g_ref[...] * (z - mu) * lax.rsqrt(var + 1e-5) + bt_ref[...]
        if relu_out:
            z = jnp.maximum(z, 0.0)
        o_ref[...] = z

    return pl.pallas_call(
        body,
        out_shape=jax.ShapeDtypeStruct((n, d), jnp.float32),
    )


def kernel(nodes, edge_indexs, graph_indicators,
           W1_0, b1_0, W2_0, b2_0, gamma_0, beta_0,
           W1_1, b1_1, W2_1, b2_1, gamma_1, beta_1,
           W1_2, b1_2, W2_2, b2_2, gamma_2, beta_2):
    del graph_indicators
    n, d = nodes.shape
    e = edge_indexs.shape[1]
    src = edge_indexs[0]
    dst = edge_indexs[1]
    params = [
        (W1_0, b1_0, W2_0, b2_0, gamma_0, beta_0),
        (W1_1, b1_1, W2_1, b2_1, gamma_1, beta_1),
        (W1_2, b1_2, W2_2, b2_2, gamma_2, beta_2),
    ]
    scatter = _make_scatter(n, d, e)
    h = nodes
    for layer, (w1, b1, w2, b2, g, bt) in enumerate(params):
        agg = scatter(h, src, dst)
        dense = _make_dense(n, w1.shape[0], d, layer < len(params) - 1)
        h = dense(h, agg[0], agg[1], w1, b1.reshape(1, d), w2, b2.reshape(1, d),
                  g.reshape(1, d), bt.reshape(1, d))
    return h


# speedup vs baseline: 3.0869x; 1.0517x over previous
import functools

import jax
import jax.numpy as jnp
from jax import lax
from jax.experimental import pallas as pl
from jax.experimental.pallas import tpu as pltpu
from jax.experimental.pallas import tpu_sc as plsc

_NC = 2
_NS = 16
_LANES = 16


@functools.lru_cache(maxsize=None)
def _make_scatter(n, d, e):
    nw = _NC * _NS
    assert e % nw == 0
    epw = e // nw
    chunk = 128
    full = epw // chunk
    tail = epw % chunk
    assert tail % 8 == 0
    rpt = (n // _NS) // 8 * 8
    rextra = n - _NS * rpt
    assert rextra % 8 == 0
    nun = 3
    assert full % nun == 0
    zsrc = chunk
    nz_full = rpt // zsrc
    zrem = rpt - nz_full * zsrc
    assert zrem % 8 == 0 and rextra <= zsrc
    mesh = plsc.VectorSubcoreMesh(core_axis_name="c", subcore_axis_name="s")

    @functools.partial(
        pl.kernel,
        mesh=mesh,
        out_type=jax.ShapeDtypeStruct((_NC, n, d), jnp.float32),
        scratch_types=(
            [pltpu.VMEM((chunk,), jnp.int32) for _ in range(nun)]
            + [pltpu.VMEM((chunk,), jnp.int32) for _ in range(nun)]
            + [pltpu.VMEM((chunk, d), jnp.float32) for _ in range(nun)]
            + [pltpu.VMEM_SHARED((n, d), jnp.float32)]
            + [pltpu.SemaphoreType.DMA for _ in range(3 * nun)]
        ),
    )
    def scatter_kernel(h_hbm, src_hbm, dst_hbm, out_hbm, *rest):
        srcv = rest[0:nun]
        dstv = rest[nun:2 * nun]
        rowsv = rest[2 * nun:3 * nun]
        acc_sh = rest[3 * nun]
        isem = rest[3 * nun + 1:3 * nun + 1 + nun]
        gsem = rest[3 * nun + 1 + nun:3 * nun + 1 + 2 * nun]
        ssem = rest[3 * nun + 1 + 2 * nun:3 * nun + 1 + 3 * nun]
        cid = lax.axis_index("c")
        sid = lax.axis_index("s")
        wid = sid * _NC + cid

        zb = rowsv[0]
        def zstore(t, carry):
            r = t // (d // _LANES)
            c16 = (t % (d // _LANES)) * _LANES
            zb[r, pl.ds(c16, _LANES)] = jnp.zeros((_LANES,), jnp.float32)
            return carry
        lax.fori_loop(0, zsrc * (d // _LANES), zstore, 0)
        row0 = sid * rpt
        def zcopy(j, carry):
            pltpu.sync_copy(zb.at[pl.ds(0, zsrc)],
                            acc_sh.at[pl.ds(row0 + j * zsrc, zsrc)])
            return carry
        lax.fori_loop(0, nz_full, zcopy, 0)
        if zrem:
            pltpu.sync_copy(zb.at[pl.ds(0, zrem)],
                            acc_sh.at[pl.ds(row0 + nz_full * zsrc, zrem)])
        @pl.when(sid == _NS - 1)
        def _ztail():
            pltpu.sync_copy(zb.at[pl.ds(0, rextra)],
                            acc_sh.at[pl.ds(_NS * rpt, rextra)])
        plsc.subcore_barrier()

        e0 = wid * epw
        def body(i, carry):
            hi = []
            for b in range(nun):
                base = e0 + (nun * i + b) * chunk
                hi.append(pltpu.async_copy(
                    src_hbm.at[pl.ds(base, chunk)], srcv[b], isem[b]))
                hi.append(pltpu.async_copy(
                    dst_hbm.at[pl.ds(base, chunk)], dstv[b], isem[b]))
            hg = []
            for b in range(nun):
                hi[2 * b].wait()
                hi[2 * b + 1].wait()
                hg.append(pltpu.async_copy(h_hbm.at[srcv[b]], rowsv[b], gsem[b]))
            hs = []
            for b in range(nun):
                hg[b].wait()
                hs.append(pltpu.async_copy(rowsv[b], acc_sh.at[dstv[b]],
                                           ssem[b], add=True))
            for b in range(nun):
                hs[b].wait()
            return carry
        lax.fori_loop(0, full // nun, body, 0)
        if tail:
            base = e0 + full * chunk
            pltpu.sync_copy(src_hbm.at[pl.ds(base, tail)], srcv[0].at[pl.ds(0, tail)])
            pltpu.sync_copy(dst_hbm.at[pl.ds(base, tail)], dstv[0].at[pl.ds(0, tail)])
            pltpu.async_copy(h_hbm.at[srcv[0].at[pl.ds(0, tail)]],
                             rowsv[0].at[pl.ds(0, tail)], gsem[0]).wait()
            pltpu.sync_copy(rowsv[0].at[pl.ds(0, tail)],
                            acc_sh.at[dstv[0].at[pl.ds(0, tail)]], add=True)
        plsc.subcore_barrier()

        pltpu.sync_copy(acc_sh.at[pl.ds(row0, rpt)], out_hbm.at[cid].at[pl.ds(row0, rpt)])
        @pl.when(sid == _NS - 1)
        def _otail():
            pltpu.sync_copy(acc_sh.at[pl.ds(_NS * rpt, rextra)],
                            out_hbm.at[cid].at[pl.ds(_NS * rpt, rextra)])

    return scatter_kernel


@functools.lru_cache(maxsize=None)
def _make_dense(n, d_in, d, relu_out):
    def body(h_ref, a0_ref, a1_ref, w1_ref, b1_ref, w2_ref, b2_ref,
             g_ref, bt_ref, o_ref):
        z = h_ref[...] + a0_ref[...] + a1_ref[...]
        z = jnp.dot(z, w1_ref[...], preferred_element_type=jnp.float32) + b1_ref[...]
        z = jnp.maximum(z, 0.0)
        z = jnp.dot(z, w2_ref[...], preferred_element_type=jnp.float32) + b2_ref[...]
        mu = jnp.mean(z, axis=0, keepdims=True)
        var = jnp.mean((z - mu) * (z - mu), axis=0, keepdims=True)
        z = g_ref[...] * (z - mu) * lax.rsqrt(var + 1e-5) + bt_ref[...]
        if relu_out:
            z = jnp.maximum(z, 0.0)
        o_ref[...] = z

    return pl.pallas_call(
        body,
        out_shape=jax.ShapeDtypeStruct((n, d), jnp.float32),
    )


def kernel(nodes, edge_indexs, graph_indicators,
           W1_0, b1_0, W2_0, b2_0, gamma_0, beta_0,
           W1_1, b1_1, W2_1, b2_1, gamma_1, beta_1,
           W1_2, b1_2, W2_2, b2_2, gamma_2, beta_2):
    del graph_indicators
    n, d = nodes.shape
    e = edge_indexs.shape[1]
    src = edge_indexs[0]
    dst = edge_indexs[1]
    params = [
        (W1_0, b1_0, W2_0, b2_0, gamma_0, beta_0),
        (W1_1, b1_1, W2_1, b2_1, gamma_1, beta_1),
        (W1_2, b1_2, W2_2, b2_2, gamma_2, beta_2),
    ]
    scatter = _make_scatter(n, d, e)
    h = nodes
    for layer, (w1, b1, w2, b2, g, bt) in enumerate(params):
        agg = scatter(h, src, dst)
        dense = _make_dense(n, w1.shape[0], d, layer < len(params) - 1)
        h = dense(h, agg[0], agg[1], w1, b1.reshape(1, d), w2, b2.reshape(1, d),
                  g.reshape(1, d), bt.reshape(1, d))
    return h
